# baseline (speedup 1.0000x reference)
import functools

import jax
import jax.numpy as jnp
from jax import lax
from jax.experimental import pallas as pl
from jax.experimental.pallas import tpu as pltpu
from jax.experimental.pallas import tpu_sc as plsc

NC = 2
NS = 16
CHUNK = 128
RC = 64
NBUF = 2


def _tc_reduce(x, Wr, br, Wg0, br_rows):
    n, ann = x.shape
    hdim = Wr.shape[1]
    nb = n // br_rows

    def body(x_ref, wr_ref, b_ref, wg_ref, h_ref, m_ref):
        h = jnp.dot(x_ref[...], wr_ref[...], preferred_element_type=jnp.float32)
        h = h + b_ref[...]
        h_ref[...] = h
        m_ref[...] = jnp.dot(h, wg_ref[...], preferred_element_type=jnp.float32)

    return pl.pallas_call(
        body,
        grid=(nb,),
        in_specs=[
            pl.BlockSpec((br_rows, ann), lambda i: (i, 0)),
            pl.BlockSpec((ann, hdim), lambda i: (0, 0)),
            pl.BlockSpec((1, hdim), lambda i: (0, 0)),
            pl.BlockSpec((hdim, hdim), lambda i: (0, 0)),
        ],
        out_specs=[
            pl.BlockSpec((br_rows, hdim), lambda i: (i, 0)),
            pl.BlockSpec((br_rows, hdim), lambda i: (i, 0)),
        ],
        out_shape=[
            jax.ShapeDtypeStruct((n, hdim), jnp.float32),
            jax.ShapeDtypeStruct((n, hdim), jnp.float32),
        ],
    )(x, Wr, br, Wg0)


def _tc_gru(h, agg, WihT, WhhT, bih, bhh, Wg_next, br_rows):
    n, hdim = h.shape
    nb = n // br_rows
    last = Wg_next is None

    def body(h_ref, agg_ref, wih_ref, whh_ref, bih_ref, bhh_ref, *rest):
        if last:
            ho_ref, = rest
        else:
            wg_ref, ho_ref, mo_ref = rest
        h_blk = h_ref[...]
        gi = jnp.dot(agg_ref[...], wih_ref[...], preferred_element_type=jnp.float32)
        gi = gi + bih_ref[...]
        gh = jnp.dot(h_blk, whh_ref[...], preferred_element_type=jnp.float32)
        gh = gh + bhh_ref[...]
        r = jax.nn.sigmoid(gi[:, :hdim] + gh[:, :hdim])
        z = jax.nn.sigmoid(gi[:, hdim:2 * hdim] + gh[:, hdim:2 * hdim])
        ng = jnp.tanh(gi[:, 2 * hdim:] + r * gh[:, 2 * hdim:])
        hn = (1.0 - z) * ng + z * h_blk
        ho_ref[...] = hn
        if not last:
            mo_ref[...] = jnp.dot(hn, wg_ref[...], preferred_element_type=jnp.float32)

    in_specs = [
        pl.BlockSpec((br_rows, hdim), lambda i: (i, 0)),
        pl.BlockSpec((br_rows, hdim), lambda i: (i, 0)),
        pl.BlockSpec((hdim, 3 * hdim), lambda i: (0, 0)),
        pl.BlockSpec((hdim, 3 * hdim), lambda i: (0, 0)),
        pl.BlockSpec((1, 3 * hdim), lambda i: (0, 0)),
        pl.BlockSpec((1, 3 * hdim), lambda i: (0, 0)),
    ]
    args = [h, agg, WihT, WhhT, bih, bhh]
    if last:
        out_specs = pl.BlockSpec((br_rows, hdim), lambda i: (i, 0))
        out_shape = jax.ShapeDtypeStruct((n, hdim), jnp.float32)
    else:
        in_specs.append(pl.BlockSpec((hdim, hdim), lambda i: (0, 0)))
        args.append(Wg_next)
        out_specs = [
            pl.BlockSpec((br_rows, hdim), lambda i: (i, 0)),
            pl.BlockSpec((br_rows, hdim), lambda i: (i, 0)),
        ]
        out_shape = [
            jax.ShapeDtypeStruct((n, hdim), jnp.float32),
            jax.ShapeDtypeStruct((n, hdim), jnp.float32),
        ]
    return pl.pallas_call(
        body,
        grid=(nb,),
        in_specs=in_specs,
        out_specs=out_specs,
        out_shape=out_shape,
    )(*args)


def _tc_head(h, Wl, bl, br_rows):
    n, hdim = h.shape
    branches = Wl.shape[1]
    nb = n // br_rows

    def body(h_ref, wl_ref, bl_ref, o_ref):
        hr = jnp.maximum(h_ref[...], 0.0)
        logit = jnp.dot(hr, wl_ref[...], preferred_element_type=jnp.float32)
        logit = logit + bl_ref[...]
        mx = jnp.max(logit, axis=1, keepdims=True)
        lse = mx + jnp.log(jnp.sum(jnp.exp(logit - mx), axis=1, keepdims=True))
        o_ref[...] = logit - lse

    return pl.pallas_call(
        body,
        grid=(nb,),
        in_specs=[
            pl.BlockSpec((br_rows, hdim), lambda i: (i, 0)),
            pl.BlockSpec((hdim, branches), lambda i: (0, 0)),
            pl.BlockSpec((1, branches), lambda i: (0, 0)),
        ],
        out_specs=pl.BlockSpec((br_rows, branches), lambda i: (i, 0)),
        out_shape=jax.ShapeDtypeStruct((n, branches), jnp.float32),
    )(h, Wl, bl)


def _sc_aggregate(m, src3, dst3, zeros_blk, n_pad):
    hdim = m.shape[1]
    cph = src3.shape[2]
    half = n_pad // NC
    rows_per_tile = half // NS
    nrow_chunks = rows_per_tile // RC
    mesh = plsc.VectorSubcoreMesh(core_axis_name="c", subcore_axis_name="s")

    @functools.partial(
        pl.kernel,
        out_type=jax.ShapeDtypeStruct((n_pad, hdim), jnp.float32),
        mesh=mesh,
        scratch_types=[
            pltpu.VMEM((cph, CHUNK), jnp.int32),
            pltpu.VMEM((cph, CHUNK), jnp.int32),
            pltpu.VMEM((nrow_chunks, RC), jnp.int32),
            pltpu.VMEM_SHARED((half, hdim), jnp.float32),
        ]
        + [pltpu.VMEM((CHUNK, hdim), jnp.float32)] * NBUF
        + [pltpu.SemaphoreType.DMA] * (2 * NBUF),
    )
    def k(m_hbm, src_hbm, dst_hbm, z_hbm, out_hbm,
          src_v, dst_v, row_v, agg_sh, *rest):
        bufs = rest[:NBUF]
        gsem = rest[NBUF:2 * NBUF]
        ssem = rest[2 * NBUF:]
        c = lax.axis_index("c")
        s = lax.axis_index("s")
        wid = c * NS + s
        pltpu.sync_copy(z_hbm, bufs[0].at[pl.ds(0, RC)])

        @pl.loop(0, nrow_chunks)
        def _(kk):
            base = s * rows_per_tile + kk * RC
            for g in range(RC // 16):
                row_v[kk, pl.ds(g * 16, 16)] = base + g * 16 + lax.iota(jnp.int32, 16)

        @pl.loop(0, nrow_chunks)
        def _(kk):
            pltpu.sync_copy(bufs[0].at[pl.ds(0, RC)], agg_sh.at[row_v.at[kk]])

        plsc.subcore_barrier()

        @pl.loop(0, 2)
        def _(hh):
            pltpu.sync_copy(src_hbm.at[wid, hh], src_v)
            pltpu.sync_copy(dst_hbm.at[wid, hh], dst_v)
            for b in range(NBUF):
                pltpu.async_copy(
                    m_hbm.at[plsc.Indices(src_v.at[b], ignored_value=-1)],
                    bufs[b], gsem[b])

            @pl.loop(0, cph, step=NBUF)
            def _(j):
                for b in range(NBUF):
                    jj = j + b
                    pltpu.make_async_copy(
                        m_hbm.at[plsc.Indices(src_v.at[jj], ignored_value=-1)],
                        bufs[b], gsem[b]).wait()
                    pltpu.async_copy(
                        bufs[b],
                        agg_sh.at[plsc.Indices(dst_v.at[jj], ignored_value=-1)],
                        ssem[b], add=True)
                for b in range(NBUF):
                    jj = j + b
                    pltpu.make_async_copy(
                        bufs[b],
                        agg_sh.at[plsc.Indices(dst_v.at[jj], ignored_value=-1)],
                        ssem[b]).wait()
                    nxt = j + NBUF + b

                    @pl.when(nxt < cph)
                    def _():
                        pltpu.async_copy(
                            m_hbm.at[plsc.Indices(src_v.at[nxt], ignored_value=-1)],
                            bufs[b], gsem[b])

        plsc.subcore_barrier()

        @pl.loop(0, nrow_chunks)
        def _(kk):
            pltpu.sync_copy(agg_sh.at[row_v.at[kk]], bufs[0].at[pl.ds(0, RC)])
            pltpu.sync_copy(
                bufs[0].at[pl.ds(0, RC)],
                out_hbm.at[pl.ds(c * half + s * rows_per_tile + kk * RC, RC)])

    return k(m, src3, dst3, zeros_blk)


def kernel(x, edge_index, W_reduce, b_reduce, ggc_weight, W_ih, W_hh, b_ih, b_hh, W_lin, b_lin):
    n, _ = x.shape
    hdim = W_reduce.shape[1]
    e = edge_index.shape[1]
    steps = ggc_weight.shape[0]

    br_rows = 400
    n_pad = -(-n // (NC * NS * RC)) * (NC * NS * RC)
    half = n_pad // NC

    ept = -(-e // (NS * CHUNK * NBUF * 2)) * (CHUNK * NBUF * 2)
    e_pad = ept * NS
    pad = e_pad - e

    src = edge_index[0]
    dst = edge_index[1]
    srcp = jnp.concatenate([src, jnp.zeros((pad,), jnp.int32)])
    dstp = jnp.concatenate([dst, jnp.full((pad,), -5, jnp.int32)])
    src_cs, dst_cs = [], []
    for cc in range(NC):
        inr = (dstp >= cc * half) & (dstp < (cc + 1) * half)
        src_cs.append(jnp.where(inr, srcp, -1))
        dst_cs.append(jnp.where(inr, dstp - cc * half, -1))
    src3 = jnp.stack(src_cs).reshape(NC * NS, 2, ept // (2 * CHUNK), CHUNK)
    dst3 = jnp.stack(dst_cs).reshape(NC * NS, 2, ept // (2 * CHUNK), CHUNK)
    zeros_blk = jnp.zeros((RC, hdim), jnp.float32)

    WihT = W_ih.T
    WhhT = W_hh.T
    bih = b_ih.reshape(1, 3 * hdim)
    bhh = b_hh.reshape(1, 3 * hdim)

    h, m = _tc_reduce(x, W_reduce, b_reduce.reshape(1, hdim), ggc_weight[0], br_rows)
    for i in range(steps):
        agg = _sc_aggregate(m, src3, dst3, zeros_blk, n_pad)
        if i + 1 < steps:
            h, m = _tc_gru(h, agg, WihT, WhhT, bih, bhh, ggc_weight[i + 1], br_rows)
        else:
            h = _tc_gru(h, agg, WihT, WhhT, bih, bhh, None, br_rows)
    return _tc_head(h, W_lin, b_lin.reshape(1, 2), br_rows)


# speedup vs baseline: 1.3303x; 1.3303x over previous
import functools

import jax
import jax.numpy as jnp
from jax import lax
from jax.experimental import pallas as pl
from jax.experimental.pallas import tpu as pltpu
from jax.experimental.pallas import tpu_sc as plsc

NC = 2
NS = 16
CHUNK = 64
RC = 64
NBUF = 4


def _tc_reduce(x, Wr, br, Wg0, br_rows):
    n, ann = x.shape
    hdim = Wr.shape[1]
    nb = n // br_rows

    def body(x_ref, wr_ref, b_ref, wg_ref, h_ref, m_ref):
        h = jnp.dot(x_ref[...], wr_ref[...], preferred_element_type=jnp.float32)
        h = h + b_ref[...]
        h_ref[...] = h
        m_ref[...] = jnp.dot(h, wg_ref[...], preferred_element_type=jnp.float32)

    return pl.pallas_call(
        body,
        grid=(nb,),
        in_specs=[
            pl.BlockSpec((br_rows, ann), lambda i: (i, 0)),
            pl.BlockSpec((ann, hdim), lambda i: (0, 0)),
            pl.BlockSpec((1, hdim), lambda i: (0, 0)),
            pl.BlockSpec((hdim, hdim), lambda i: (0, 0)),
        ],
        out_specs=[
            pl.BlockSpec((br_rows, hdim), lambda i: (i, 0)),
            pl.BlockSpec((br_rows, hdim), lambda i: (i, 0)),
        ],
        out_shape=[
            jax.ShapeDtypeStruct((n, hdim), jnp.float32),
            jax.ShapeDtypeStruct((n, hdim), jnp.float32),
        ],
    )(x, Wr, br, Wg0)


def _tc_gru(h, agg, WihT, WhhT, bih, bhh, Wg_next, br_rows):
    n, hdim = h.shape
    nb = n // br_rows
    last = Wg_next is None

    def body(h_ref, agg_ref, wih_ref, whh_ref, bih_ref, bhh_ref, *rest):
        if last:
            ho_ref, = rest
        else:
            wg_ref, ho_ref, mo_ref = rest
        h_blk = h_ref[...]
        gi = jnp.dot(agg_ref[...], wih_ref[...], preferred_element_type=jnp.float32)
        gi = gi + bih_ref[...]
        gh = jnp.dot(h_blk, whh_ref[...], preferred_element_type=jnp.float32)
        gh = gh + bhh_ref[...]
        r = jax.nn.sigmoid(gi[:, :hdim] + gh[:, :hdim])
        z = jax.nn.sigmoid(gi[:, hdim:2 * hdim] + gh[:, hdim:2 * hdim])
        ng = jnp.tanh(gi[:, 2 * hdim:] + r * gh[:, 2 * hdim:])
        hn = (1.0 - z) * ng + z * h_blk
        ho_ref[...] = hn
        if not last:
            mo_ref[...] = jnp.dot(hn, wg_ref[...], preferred_element_type=jnp.float32)

    in_specs = [
        pl.BlockSpec((br_rows, hdim), lambda i: (i, 0)),
        pl.BlockSpec((br_rows, hdim), lambda i: (i, 0)),
        pl.BlockSpec((hdim, 3 * hdim), lambda i: (0, 0)),
        pl.BlockSpec((hdim, 3 * hdim), lambda i: (0, 0)),
        pl.BlockSpec((1, 3 * hdim), lambda i: (0, 0)),
        pl.BlockSpec((1, 3 * hdim), lambda i: (0, 0)),
    ]
    args = [h, agg, WihT, WhhT, bih, bhh]
    if last:
        out_specs = pl.BlockSpec((br_rows, hdim), lambda i: (i, 0))
        out_shape = jax.ShapeDtypeStruct((n, hdim), jnp.float32)
    else:
        in_specs.append(pl.BlockSpec((hdim, hdim), lambda i: (0, 0)))
        args.append(Wg_next)
        out_specs = [
            pl.BlockSpec((br_rows, hdim), lambda i: (i, 0)),
            pl.BlockSpec((br_rows, hdim), lambda i: (i, 0)),
        ]
        out_shape = [
            jax.ShapeDtypeStruct((n, hdim), jnp.float32),
            jax.ShapeDtypeStruct((n, hdim), jnp.float32),
        ]
    return pl.pallas_call(
        body,
        grid=(nb,),
        in_specs=in_specs,
        out_specs=out_specs,
        out_shape=out_shape,
    )(*args)


def _tc_head(h, Wl, bl, br_rows):
    n, hdim = h.shape
    branches = Wl.shape[1]
    nb = n // br_rows

    def body(h_ref, wl_ref, bl_ref, o_ref):
        hr = jnp.maximum(h_ref[...], 0.0)
        logit = jnp.dot(hr, wl_ref[...], preferred_element_type=jnp.float32)
        logit = logit + bl_ref[...]
        mx = jnp.max(logit, axis=1, keepdims=True)
        lse = mx + jnp.log(jnp.sum(jnp.exp(logit - mx), axis=1, keepdims=True))
        o_ref[...] = logit - lse

    return pl.pallas_call(
        body,
        grid=(nb,),
        in_specs=[
            pl.BlockSpec((br_rows, hdim), lambda i: (i, 0)),
            pl.BlockSpec((hdim, branches), lambda i: (0, 0)),
            pl.BlockSpec((1, branches), lambda i: (0, 0)),
        ],
        out_specs=pl.BlockSpec((br_rows, branches), lambda i: (i, 0)),
        out_shape=jax.ShapeDtypeStruct((n, branches), jnp.float32),
    )(h, Wl, bl)


def _sc_aggregate(m, src3, dst3, zeros_blk, n_pad):
    hdim = m.shape[1]
    cph = src3.shape[2]
    half = n_pad // NC
    rows_per_tile = half // NS
    nrow_chunks = rows_per_tile // RC
    mesh = plsc.VectorSubcoreMesh(core_axis_name="c", subcore_axis_name="s")

    @functools.partial(
        pl.kernel,
        out_type=jax.ShapeDtypeStruct((n_pad, hdim), jnp.float32),
        mesh=mesh,
        scratch_types=[
            pltpu.VMEM((cph, CHUNK), jnp.int32),
            pltpu.VMEM((cph, CHUNK), jnp.int32),
            pltpu.VMEM((nrow_chunks, RC), jnp.int32),
            pltpu.VMEM_SHARED((half, hdim), jnp.float32),
        ]
        + [pltpu.VMEM((CHUNK, hdim), jnp.float32)] * NBUF
        + [pltpu.SemaphoreType.DMA] * (2 * NBUF),
    )
    def k(m_hbm, src_hbm, dst_hbm, z_hbm, out_hbm,
          src_v, dst_v, row_v, agg_sh, *rest):
        bufs = rest[:NBUF]
        gsem = rest[NBUF:2 * NBUF]
        ssem = rest[2 * NBUF:]
        c = lax.axis_index("c")
        s = lax.axis_index("s")
        wid = c * NS + s
        pltpu.sync_copy(z_hbm, bufs[0].at[pl.ds(0, RC)])

        @pl.loop(0, nrow_chunks)
        def _(kk):
            base = s * rows_per_tile + kk * RC
            for g in range(RC // 16):
                row_v[kk, pl.ds(g * 16, 16)] = base + g * 16 + lax.iota(jnp.int32, 16)

        @pl.loop(0, nrow_chunks)
        def _(kk):
            pltpu.sync_copy(bufs[0].at[pl.ds(0, RC)], agg_sh.at[row_v.at[kk]])

        plsc.subcore_barrier()

        @pl.loop(0, 2)
        def _(hh):
            pltpu.sync_copy(src_hbm.at[wid, hh], src_v)
            pltpu.sync_copy(dst_hbm.at[wid, hh], dst_v)
            for b in range(NBUF):
                pltpu.async_copy(
                    m_hbm.at[plsc.Indices(src_v.at[b], ignored_value=-1)],
                    bufs[b], gsem[b])

            @pl.loop(0, cph, step=NBUF)
            def _(j):
                for b in range(NBUF):
                    jj = j + b
                    pltpu.make_async_copy(
                        m_hbm.at[plsc.Indices(src_v.at[jj], ignored_value=-1)],
                        bufs[b], gsem[b]).wait()
                    pltpu.async_copy(
                        bufs[b],
                        agg_sh.at[plsc.Indices(dst_v.at[jj], ignored_value=-1)],
                        ssem[b], add=True)
                for b in range(NBUF):
                    jj = j + b
                    pltpu.make_async_copy(
                        bufs[b],
                        agg_sh.at[plsc.Indices(dst_v.at[jj], ignored_value=-1)],
                        ssem[b]).wait()
                    nxt = j + NBUF + b

                    @pl.when(nxt < cph)
                    def _():
                        pltpu.async_copy(
                            m_hbm.at[plsc.Indices(src_v.at[nxt], ignored_value=-1)],
                            bufs[b], gsem[b])

        plsc.subcore_barrier()

        @pl.loop(0, nrow_chunks)
        def _(kk):
            pltpu.sync_copy(agg_sh.at[row_v.at[kk]], bufs[0].at[pl.ds(0, RC)])
            pltpu.sync_copy(
                bufs[0].at[pl.ds(0, RC)],
                out_hbm.at[pl.ds(c * half + s * rows_per_tile + kk * RC, RC)])

    return k(m, src3, dst3, zeros_blk)


def kernel(x, edge_index, W_reduce, b_reduce, ggc_weight, W_ih, W_hh, b_ih, b_hh, W_lin, b_lin):
    n, _ = x.shape
    hdim = W_reduce.shape[1]
    e = edge_index.shape[1]
    steps = ggc_weight.shape[0]

    br_rows = 400
    n_pad = -(-n // (NC * NS * RC)) * (NC * NS * RC)
    half = n_pad // NC

    ept = -(-e // (NS * CHUNK * NBUF * 2)) * (CHUNK * NBUF * 2)
    e_pad = ept * NS
    pad = e_pad - e

    src = edge_index[0]
    dst = edge_index[1]
    srcp = jnp.concatenate([src, jnp.zeros((pad,), jnp.int32)])
    dstp = jnp.concatenate([dst, jnp.full((pad,), -5, jnp.int32)])
    src_cs, dst_cs = [], []
    for cc in range(NC):
        inr = (dstp >= cc * half) & (dstp < (cc + 1) * half)
        src_cs.append(jnp.where(inr, srcp, -1))
        dst_cs.append(jnp.where(inr, dstp - cc * half, -1))
    src3 = jnp.stack(src_cs).reshape(NC * NS, 2, ept // (2 * CHUNK), CHUNK)
    dst3 = jnp.stack(dst_cs).reshape(NC * NS, 2, ept // (2 * CHUNK), CHUNK)
    zeros_blk = jnp.zeros((RC, hdim), jnp.float32)

    WihT = W_ih.T
    WhhT = W_hh.T
    bih = b_ih.reshape(1, 3 * hdim)
    bhh = b_hh.reshape(1, 3 * hdim)

    h, m = _tc_reduce(x, W_reduce, b_reduce.reshape(1, hdim), ggc_weight[0], br_rows)
    for i in range(steps):
        agg = _sc_aggregate(m, src3, dst3, zeros_blk, n_pad)
        if i + 1 < steps:
            h, m = _tc_gru(h, agg, WihT, WhhT, bih, bhh, ggc_weight[i + 1], br_rows)
        else:
            h = _tc_gru(h, agg, WihT, WhhT, bih, bhh, None, br_rows)
    return _tc_head(h, W_lin, b_lin.reshape(1, 2), br_rows)
